# Initial kernel scaffold; baseline (speedup 1.0000x reference)
import functools

import jax
import jax.numpy as jnp
from jax import lax
from jax.experimental import pallas as pl
from jax.experimental.pallas import tpu as pltpu
from jax.experimental.pallas import tpu_sc as plsc

N_NODES = 10000
D = 128
NC = 2
NS = 16
NW = NC * NS
CHUNK = 128
ACC_ROWS = 10240
ROWS_PER_SUB = ACC_ROWS // NS
WB_CHUNKS = ROWS_PER_SUB // CHUNK


def _sc_aggregate(x, src, dst, n_chunks):
    mesh = plsc.VectorSubcoreMesh(core_axis_name="c", subcore_axis_name="s")

    @functools.partial(
        pl.kernel,
        out_type=(
            jax.ShapeDtypeStruct((NC, ACC_ROWS, D), jnp.float32),
            jax.ShapeDtypeStruct((NC, ACC_ROWS, 16), jnp.float32),
        ),
        mesh=mesh,
        scratch_types=[
            pltpu.VMEM((CHUNK,), jnp.int32),
            pltpu.VMEM((CHUNK,), jnp.int32),
            pltpu.VMEM((CHUNK, D), jnp.float32),
            pltpu.VMEM((CHUNK, 16), jnp.float32),
            pltpu.VMEM((CHUNK, 16), jnp.float32),
            pltpu.VMEM_SHARED((ACC_ROWS, D), jnp.float32),
            pltpu.VMEM_SHARED((ACC_ROWS, 16), jnp.float32),
            pltpu.SemaphoreType.DMA,
        ],
    )
    def k(x_hbm, src_hbm, dst_hbm, part_hbm, cnt_hbm,
          sidx_v, didx_v, rows_v, ones_v, z16_v, acc_sh, cntacc_sh, sem):
        cid = lax.axis_index("c")
        sid = lax.axis_index("s")
        wid = cid * NS + sid
        base = wid * (n_chunks * CHUNK)
        sub_row0 = sid * ROWS_PER_SUB

        def fill(i, _):
            for c in range(D // 16):
                rows_v[i, pl.ds(c * 16, 16)] = jnp.zeros((16,), jnp.float32)
            ones_v[i, :] = jnp.ones((16,), jnp.float32)
            z16_v[i, :] = jnp.zeros((16,), jnp.float32)
            return 0

        lax.fori_loop(0, CHUNK, fill, 0)

        def zero(c, _):
            r0 = sub_row0 + c * CHUNK
            pltpu.sync_copy(rows_v, acc_sh.at[pl.ds(r0, CHUNK)])
            pltpu.sync_copy(z16_v, cntacc_sh.at[pl.ds(r0, CHUNK)])
            return 0

        lax.fori_loop(0, WB_CHUNKS, zero, 0)
        plsc.subcore_barrier()

        def step(kk, _):
            off = base + kk * CHUNK
            pltpu.sync_copy(src_hbm.at[pl.ds(off, CHUNK)], sidx_v)
            pltpu.sync_copy(dst_hbm.at[pl.ds(off, CHUNK)], didx_v)
            pltpu.async_copy(x_hbm.at[sidx_v], rows_v, sem).wait()
            pltpu.sync_copy(rows_v, acc_sh.at[didx_v], add=True)
            pltpu.sync_copy(ones_v, cntacc_sh.at[didx_v], add=True)
            return 0

        lax.fori_loop(0, n_chunks, step, 0)
        plsc.subcore_barrier()

        def wb(c, _):
            r0 = sub_row0 + c * CHUNK
            pltpu.sync_copy(acc_sh.at[pl.ds(r0, CHUNK)], rows_v)
            pltpu.sync_copy(rows_v, part_hbm.at[cid, pl.ds(r0, CHUNK)])
            pltpu.sync_copy(cntacc_sh.at[pl.ds(r0, CHUNK)], z16_v)
            pltpu.sync_copy(z16_v, cnt_hbm.at[cid, pl.ds(r0, CHUNK)])
            return 0

        lax.fori_loop(0, WB_CHUNKS, wb, 0)

    return k(x, src, dst)


def _tc_finish_body(p_ref, c_ref, x_ref, w_ref, o_ref):
    psum = p_ref[0] + p_ref[1]
    cnt = c_ref[0, :, 0:1] + c_ref[1, :, 0:1]
    mean = psum / jnp.maximum(cnt, 1.0)
    y = mean + x_ref[...]
    o_ref[...] = lax.dot_general(
        y, w_ref[...], (((1,), (1,)), ((), ())),
        preferred_element_type=jnp.float32)


def _tc_finish(part, cnt, x, w_r):
    blk = 1000
    grid = (N_NODES // blk,)
    return pl.pallas_call(
        _tc_finish_body,
        grid=grid,
        in_specs=[
            pl.BlockSpec((NC, blk, D), lambda i: (0, i, 0)),
            pl.BlockSpec((NC, blk, 16), lambda i: (0, i, 0)),
            pl.BlockSpec((blk, D), lambda i: (i, 0)),
            pl.BlockSpec((D, D), lambda i: (0, 0)),
        ],
        out_specs=pl.BlockSpec((blk, D), lambda i: (i, 0)),
        out_shape=jax.ShapeDtypeStruct((N_NODES, D), jnp.float32),
    )(part, cnt, x, w_r)


@jax.jit
def kernel(x, edge_index, W_r):
    e = edge_index.shape[1]
    n_chunks = -(-e // (NW * CHUNK))
    e_pad = NW * n_chunks * CHUNK
    dst = edge_index[0].astype(jnp.int32)
    src = edge_index[1].astype(jnp.int32)
    pad = e_pad - e
    if pad:
        src = jnp.concatenate([src, jnp.zeros((pad,), jnp.int32)])
        dst = jnp.concatenate([dst, jnp.full((pad,), N_NODES, jnp.int32)])
    part, cnt = _sc_aggregate(x, src, dst, n_chunks)
    return _tc_finish(part, cnt, x, W_r)


# speedup vs baseline: 4.7715x; 4.7715x over previous
import functools

import jax
import jax.numpy as jnp
from jax import lax
from jax.experimental import pallas as pl
from jax.experimental.pallas import tpu as pltpu
from jax.experimental.pallas import tpu_sc as plsc

N_NODES = 10000
D = 128
DH = D // 2
NC = 2
NS = 16
CHUNK = 128
ACC_ROWS = 10112
ROWS_PER_SUB = ACC_ROWS // NS
WB_OFFS = (0, 128, 256, 384, ROWS_PER_SUB - CHUNK)


def _sc_aggregate(xh, src, dst, n_chunks):
    mesh = plsc.VectorSubcoreMesh(core_axis_name="c", subcore_axis_name="s")

    @functools.partial(
        pl.kernel,
        out_type=(
            jax.ShapeDtypeStruct((NC, ACC_ROWS, DH), jnp.float32),
            jax.ShapeDtypeStruct((ACC_ROWS, 16), jnp.float32),
        ),
        mesh=mesh,
        compiler_params=pltpu.CompilerParams(use_tc_tiling_on_sc=False),
        scratch_types=[
            pltpu.VMEM((CHUNK,), jnp.int32),
            pltpu.VMEM((CHUNK,), jnp.int32),
            pltpu.VMEM((CHUNK, DH), jnp.float32),
            pltpu.VMEM((CHUNK, 16), jnp.float32),
            pltpu.VMEM((CHUNK, 16), jnp.float32),
            pltpu.VMEM_SHARED((ACC_ROWS, DH), jnp.float32),
            pltpu.VMEM_SHARED((ACC_ROWS, 16), jnp.float32),
            pltpu.SemaphoreType.DMA,
        ],
    )
    def k(xh_hbm, src_hbm, dst_hbm, part_hbm, cnt_hbm,
          sidx_v, didx_v, rows_v, ones_v, z16_v, acc_sh, cntacc_sh, sem):
        cid = lax.axis_index("c")
        sid = lax.axis_index("s")
        base = sid * (n_chunks * CHUNK)
        sub_row0 = sid * ROWS_PER_SUB
        row_off = cid.astype(jnp.int32) * N_NODES

        def fill(i, _):
            for c in range(DH // 16):
                rows_v[i, pl.ds(c * 16, 16)] = jnp.zeros((16,), jnp.float32)
            ones_v[i, :] = jnp.ones((16,), jnp.float32)
            z16_v[i, :] = jnp.zeros((16,), jnp.float32)
            return 0

        lax.fori_loop(0, CHUNK, fill, 0)

        for woff in WB_OFFS:
            r0 = sub_row0 + woff
            pltpu.sync_copy(rows_v, acc_sh.at[pl.ds(r0, CHUNK)])
            pltpu.sync_copy(z16_v, cntacc_sh.at[pl.ds(r0, CHUNK)])
        plsc.subcore_barrier()

        def step(kk, _):
            off = base + kk * CHUNK
            pltpu.sync_copy(src_hbm.at[pl.ds(off, CHUNK)], sidx_v)
            pltpu.sync_copy(dst_hbm.at[pl.ds(off, CHUNK)], didx_v)
            for c in range(CHUNK // 16):
                sl = pl.ds(c * 16, 16)
                sidx_v[sl] = sidx_v[sl] + row_off
            pltpu.async_copy(xh_hbm.at[sidx_v], rows_v, sem).wait()
            pltpu.sync_copy(rows_v, acc_sh.at[didx_v], add=True)
            pltpu.sync_copy(ones_v, cntacc_sh.at[didx_v], add=True)
            return 0

        lax.fori_loop(0, n_chunks, step, 0)
        plsc.subcore_barrier()

        for woff in WB_OFFS:
            r0 = sub_row0 + woff
            pltpu.sync_copy(acc_sh.at[pl.ds(r0, CHUNK)], rows_v)
            pltpu.sync_copy(rows_v, part_hbm.at[cid, pl.ds(r0, CHUNK)])

        @pl.when(cid == 0)
        def _():
            for woff in WB_OFFS:
                r0 = sub_row0 + woff
                pltpu.sync_copy(cntacc_sh.at[pl.ds(r0, CHUNK)], z16_v)
                pltpu.sync_copy(z16_v, cnt_hbm.at[pl.ds(r0, CHUNK)])

    return k(xh, src, dst)


def _tc_finish_body(p_ref, c_ref, x_ref, w_ref, o_ref):
    cnt = jnp.maximum(c_ref[:, 0:1], 1.0)
    y_lo = p_ref[0] / cnt + x_ref[:, :DH]
    y_hi = p_ref[1] / cnt + x_ref[:, DH:]
    o_ref[...] = lax.dot_general(
        y_lo, w_ref[:, :DH], (((1,), (1,)), ((), ())),
        preferred_element_type=jnp.float32) + lax.dot_general(
        y_hi, w_ref[:, DH:], (((1,), (1,)), ((), ())),
        preferred_element_type=jnp.float32)


def _tc_finish(part, cnt, x, w_r):
    blk = 1000
    grid = (N_NODES // blk,)
    return pl.pallas_call(
        _tc_finish_body,
        grid=grid,
        in_specs=[
            pl.BlockSpec((NC, blk, DH), lambda i: (0, i, 0)),
            pl.BlockSpec((blk, 16), lambda i: (i, 0)),
            pl.BlockSpec((blk, D), lambda i: (i, 0)),
            pl.BlockSpec((D, D), lambda i: (0, 0)),
        ],
        out_specs=pl.BlockSpec((blk, D), lambda i: (i, 0)),
        out_shape=jax.ShapeDtypeStruct((N_NODES, D), jnp.float32),
    )(part, cnt, x, w_r)


@jax.jit
def kernel(x, edge_index, W_r):
    e = edge_index.shape[1]
    n_chunks = -(-e // (NS * CHUNK))
    e_pad = NS * n_chunks * CHUNK
    dst = edge_index[0].astype(jnp.int32)
    src = edge_index[1].astype(jnp.int32)
    pad = e_pad - e
    if pad:
        src = jnp.concatenate([src, jnp.zeros((pad,), jnp.int32)])
        dst = jnp.concatenate([dst, jnp.full((pad,), N_NODES, jnp.int32)])
    xh = jnp.concatenate([x[:, :DH], x[:, DH:]], axis=0)
    part, cnt = _sc_aggregate(xh, src, dst, n_chunks)
    return _tc_finish(part, cnt, x, W_r)


# speedup vs baseline: 5.2394x; 1.0981x over previous
import functools

import jax
import jax.numpy as jnp
from jax import lax
from jax.experimental import pallas as pl
from jax.experimental.pallas import tpu as pltpu
from jax.experimental.pallas import tpu_sc as plsc

N_NODES = 10000
D = 128
DH = D // 2
NC = 2
NS = 16
CHUNK = 128
NB = 8
SB = NB * CHUNK
ACC_ROWS = 10112
ROWS_PER_SUB = ACC_ROWS // NS
WB_OFFS = (0, 128, 256, 384, ROWS_PER_SUB - CHUNK)


def _sc_aggregate(xh, src2, dst2, n_sb):
    mesh = plsc.VectorSubcoreMesh(core_axis_name="c", subcore_axis_name="s")

    @functools.partial(
        pl.kernel,
        out_type=(
            jax.ShapeDtypeStruct((NC, ACC_ROWS, DH), jnp.float32),
            jax.ShapeDtypeStruct((ACC_ROWS, 16), jnp.float32),
        ),
        mesh=mesh,
        compiler_params=pltpu.CompilerParams(use_tc_tiling_on_sc=False),
        scratch_types=[
            pltpu.VMEM((NB, CHUNK), jnp.int32),
            pltpu.VMEM((NB, CHUNK), jnp.int32),
            pltpu.VMEM((NB, CHUNK, DH), jnp.float32),
            pltpu.VMEM((CHUNK, 16), jnp.float32),
            pltpu.VMEM((CHUNK, 16), jnp.float32),
            pltpu.VMEM_SHARED((ACC_ROWS, DH), jnp.float32),
            pltpu.VMEM_SHARED((ACC_ROWS, 16), jnp.float32),
            pltpu.SemaphoreType.DMA((NB,)),
            pltpu.SemaphoreType.DMA((NB,)),
        ],
    )
    def k(xh_hbm, src_hbm, dst_hbm, part_hbm, cnt_hbm,
          sidx_v, didx_v, rows_v, ones_v, z16_v, acc_sh, cntacc_sh,
          gsem, ssem):
        cid = lax.axis_index("c")
        sid = lax.axis_index("s")
        sub_row0 = sid * ROWS_PER_SUB

        def fill(i, _):
            for c in range(DH // 16):
                rows_v[0, i, pl.ds(c * 16, 16)] = jnp.zeros((16,), jnp.float32)
            ones_v[i, :] = jnp.ones((16,), jnp.float32)
            z16_v[i, :] = jnp.zeros((16,), jnp.float32)
            return 0

        lax.fori_loop(0, CHUNK, fill, 0)

        for woff in WB_OFFS:
            r0 = sub_row0 + woff
            pltpu.sync_copy(rows_v.at[0], acc_sh.at[pl.ds(r0, CHUNK)])
            pltpu.sync_copy(z16_v, cntacc_sh.at[pl.ds(r0, CHUNK)])
        plsc.subcore_barrier()

        def step(b, _):
            sb = sid * n_sb + b
            pltpu.sync_copy(src_hbm.at[cid, sb], sidx_v)
            pltpu.sync_copy(dst_hbm.at[sb], didx_v)
            for j in range(NB):
                pltpu.async_copy(xh_hbm.at[sidx_v.at[j]], rows_v.at[j],
                                 gsem.at[j])
            for j in range(NB):
                pltpu.make_async_copy(xh_hbm.at[sidx_v.at[j]], rows_v.at[j],
                                      gsem.at[j]).wait()
                pltpu.async_copy(rows_v.at[j], acc_sh.at[didx_v.at[j]],
                                 ssem.at[j], add=True)
                pltpu.sync_copy(ones_v, cntacc_sh.at[didx_v.at[j]], add=True)
            for j in range(NB):
                pltpu.make_async_copy(rows_v.at[j], acc_sh.at[didx_v.at[j]],
                                      ssem.at[j]).wait()
            return 0

        lax.fori_loop(0, n_sb, step, 0)
        plsc.subcore_barrier()

        for woff in WB_OFFS:
            r0 = sub_row0 + woff
            pltpu.sync_copy(acc_sh.at[pl.ds(r0, CHUNK)], rows_v.at[0])
            pltpu.sync_copy(rows_v.at[0], part_hbm.at[cid, pl.ds(r0, CHUNK)])

        @pl.when(cid == 0)
        def _():
            for woff in WB_OFFS:
                r0 = sub_row0 + woff
                pltpu.sync_copy(cntacc_sh.at[pl.ds(r0, CHUNK)], z16_v)
                pltpu.sync_copy(z16_v, cnt_hbm.at[pl.ds(r0, CHUNK)])

    return k(xh, src2, dst2)


def _tc_finish_body(p_ref, c_ref, x_ref, w_ref, o_ref):
    cnt = jnp.maximum(c_ref[:, 0:1], 1.0)
    y_lo = p_ref[0] / cnt + x_ref[:, :DH]
    y_hi = p_ref[1] / cnt + x_ref[:, DH:]
    o_ref[...] = lax.dot_general(
        y_lo, w_ref[:, :DH], (((1,), (1,)), ((), ())),
        preferred_element_type=jnp.float32) + lax.dot_general(
        y_hi, w_ref[:, DH:], (((1,), (1,)), ((), ())),
        preferred_element_type=jnp.float32)


def _tc_finish(part, cnt, x, w_r):
    blk = 1000
    grid = (N_NODES // blk,)
    return pl.pallas_call(
        _tc_finish_body,
        grid=grid,
        in_specs=[
            pl.BlockSpec((NC, blk, DH), lambda i: (0, i, 0)),
            pl.BlockSpec((blk, 16), lambda i: (i, 0)),
            pl.BlockSpec((blk, D), lambda i: (i, 0)),
            pl.BlockSpec((D, D), lambda i: (0, 0)),
        ],
        out_specs=pl.BlockSpec((blk, D), lambda i: (i, 0)),
        out_shape=jax.ShapeDtypeStruct((N_NODES, D), jnp.float32),
    )(part, cnt, x, w_r)


@jax.jit
def kernel(x, edge_index, W_r):
    e = edge_index.shape[1]
    n_sb = -(-e // (NS * SB))
    e_pad = NS * n_sb * SB
    dst = edge_index[0].astype(jnp.int32)
    src = edge_index[1].astype(jnp.int32)
    pad = e_pad - e
    if pad:
        src = jnp.concatenate([src, jnp.zeros((pad,), jnp.int32)])
        dst = jnp.concatenate([dst, jnp.full((pad,), N_NODES, jnp.int32)])
    src2 = jnp.stack([src, src + N_NODES]).reshape(NC, NS * n_sb, NB, CHUNK)
    dst2 = dst.reshape(NS * n_sb, NB, CHUNK)
    xh = jnp.concatenate([x[:, :DH], x[:, DH:]], axis=0)
    part, cnt = _sc_aggregate(xh, src2, dst2, n_sb)
    return _tc_finish(part, cnt, x, W_r)


# speedup vs baseline: 5.7085x; 1.0895x over previous
import functools

import jax
import jax.numpy as jnp
from jax import lax
from jax.experimental import pallas as pl
from jax.experimental.pallas import tpu as pltpu
from jax.experimental.pallas import tpu_sc as plsc

N_NODES = 10000
D = 128
DH = D // 2
NC = 2
NS = 16
CHUNK = 128
NB = 8
SB = NB * CHUNK
ACC_ROWS = 10112
ROWS_PER_SUB = ACC_ROWS // NS
WB_OFFS = (0, 128, 256, 384, ROWS_PER_SUB - CHUNK)


def _sc_aggregate(xh, src2, dst2, n_sb):
    mesh = plsc.VectorSubcoreMesh(core_axis_name="c", subcore_axis_name="s")

    @functools.partial(
        pl.kernel,
        out_type=(
            jax.ShapeDtypeStruct((NC, ACC_ROWS, DH), jnp.float32),
            jax.ShapeDtypeStruct((NC, ACC_ROWS, 16), jnp.float32),
        ),
        mesh=mesh,
        compiler_params=pltpu.CompilerParams(use_tc_tiling_on_sc=False),
        scratch_types=[
            pltpu.VMEM((2, NB, CHUNK), jnp.int32),
            pltpu.VMEM((2, NB, CHUNK), jnp.int32),
            pltpu.VMEM((NB, CHUNK, DH), jnp.float32),
            pltpu.VMEM((CHUNK, 16), jnp.float32),
            pltpu.VMEM((CHUNK, 16), jnp.float32),
            pltpu.VMEM_SHARED((ACC_ROWS, DH), jnp.float32),
            pltpu.VMEM_SHARED((ACC_ROWS, 16), jnp.float32),
            pltpu.SemaphoreType.DMA((NB,)),
            pltpu.SemaphoreType.DMA((NB,)),
            pltpu.SemaphoreType.DMA((NB,)),
            pltpu.SemaphoreType.DMA((2,)),
        ],
    )
    def k(xh_hbm, src_hbm, dst_hbm, part_hbm, cnt_hbm,
          sidx_v, didx_v, rows_v, ones_v, z16_v, acc_sh, cntacc_sh,
          gsem, ssem, csem, isem):
        cid = lax.axis_index("c")
        sid = lax.axis_index("s")
        sub_row0 = sid * ROWS_PER_SUB
        half = (n_sb + 1) // 2

        def fill(i, _):
            for c in range(DH // 16):
                rows_v[0, i, pl.ds(c * 16, 16)] = jnp.zeros((16,), jnp.float32)
            ones_v[i, :] = jnp.ones((16,), jnp.float32)
            z16_v[i, :] = jnp.zeros((16,), jnp.float32)
            return 0

        lax.fori_loop(0, CHUNK, fill, 0)

        for woff in WB_OFFS:
            r0 = sub_row0 + woff
            pltpu.sync_copy(rows_v.at[0], acc_sh.at[pl.ds(r0, CHUNK)])
            pltpu.sync_copy(z16_v, cntacc_sh.at[pl.ds(r0, CHUNK)])
        plsc.subcore_barrier()

        def prefetch(b, slot):
            sb = sid * n_sb + b
            pltpu.async_copy(src_hbm.at[cid, sb], sidx_v.at[slot],
                             isem.at[slot])
            pltpu.async_copy(dst_hbm.at[sb], didx_v.at[slot], isem.at[slot])

        def wait_idx(slot):
            pltpu.make_async_copy(src_hbm.at[cid, 0], sidx_v.at[slot],
                                  isem.at[slot]).wait()
            pltpu.make_async_copy(dst_hbm.at[0], didx_v.at[slot],
                                  isem.at[slot]).wait()

        def drain_rows(slot):
            for j in range(NB):
                pltpu.make_async_copy(rows_v.at[j],
                                      acc_sh.at[didx_v.at[slot, j]],
                                      ssem.at[j]).wait()

        def drain_cnt(slot):
            for j in range(NB):
                pltpu.make_async_copy(ones_v,
                                      cntacc_sh.at[didx_v.at[slot, j]],
                                      csem.at[j]).wait()

        def counts_at(b):
            return lax.select(cid == 0, b < half, b >= half)

        prefetch(0, 0)

        def step(b, _):
            slot = lax.rem(b, 2)
            @pl.when(b > 0)
            def _():
                drain_rows(1 - slot)

            @pl.when((b > 0) & counts_at(b - 1))
            def _():
                drain_cnt(1 - slot)
            wait_idx(slot)

            @pl.when(b + 1 < n_sb)
            def _():
                prefetch(b + 1, 1 - slot)

            for j in range(NB):
                pltpu.async_copy(xh_hbm.at[sidx_v.at[slot, j]], rows_v.at[j],
                                 gsem.at[j])
            count_here = counts_at(b)
            for j in range(NB):
                pltpu.make_async_copy(xh_hbm.at[sidx_v.at[slot, j]],
                                      rows_v.at[j], gsem.at[j]).wait()
                pltpu.async_copy(rows_v.at[j], acc_sh.at[didx_v.at[slot, j]],
                                 ssem.at[j], add=True)

                @pl.when(count_here)
                def _():
                    pltpu.async_copy(ones_v, cntacc_sh.at[didx_v.at[slot, j]],
                                     csem.at[j], add=True)
            return 0

        lax.fori_loop(0, n_sb, step, 0)
        drain_rows((n_sb - 1) % 2)

        @pl.when(counts_at(n_sb - 1))
        def _():
            drain_cnt((n_sb - 1) % 2)
        plsc.subcore_barrier()

        for woff in WB_OFFS:
            r0 = sub_row0 + woff
            pltpu.sync_copy(acc_sh.at[pl.ds(r0, CHUNK)], rows_v.at[0])
            pltpu.sync_copy(rows_v.at[0], part_hbm.at[cid, pl.ds(r0, CHUNK)])
            pltpu.sync_copy(cntacc_sh.at[pl.ds(r0, CHUNK)], z16_v)
            pltpu.sync_copy(z16_v, cnt_hbm.at[cid, pl.ds(r0, CHUNK)])

    return k(xh, src2, dst2)


def _tc_finish_body(p_ref, c_ref, x_ref, w_ref, o_ref):
    cnt = jnp.maximum(c_ref[0, :, 0:1] + c_ref[1, :, 0:1], 1.0)
    y_lo = p_ref[0] / cnt + x_ref[:, :DH]
    y_hi = p_ref[1] / cnt + x_ref[:, DH:]
    o_ref[...] = lax.dot_general(
        y_lo, w_ref[:, :DH], (((1,), (1,)), ((), ())),
        preferred_element_type=jnp.float32) + lax.dot_general(
        y_hi, w_ref[:, DH:], (((1,), (1,)), ((), ())),
        preferred_element_type=jnp.float32)


def _tc_finish(part, cnt, x, w_r):
    blk = 1000
    grid = (N_NODES // blk,)
    return pl.pallas_call(
        _tc_finish_body,
        grid=grid,
        in_specs=[
            pl.BlockSpec((NC, blk, DH), lambda i: (0, i, 0)),
            pl.BlockSpec((NC, blk, 16), lambda i: (0, i, 0)),
            pl.BlockSpec((blk, D), lambda i: (i, 0)),
            pl.BlockSpec((D, D), lambda i: (0, 0)),
        ],
        out_specs=pl.BlockSpec((blk, D), lambda i: (i, 0)),
        out_shape=jax.ShapeDtypeStruct((N_NODES, D), jnp.float32),
    )(part, cnt, x, w_r)


@jax.jit
def kernel(x, edge_index, W_r):
    e = edge_index.shape[1]
    n_sb = -(-e // (NS * SB))
    e_pad = NS * n_sb * SB
    dst = edge_index[0].astype(jnp.int32)
    src = edge_index[1].astype(jnp.int32)
    pad = e_pad - e
    if pad:
        src = jnp.concatenate([src, jnp.zeros((pad,), jnp.int32)])
        dst = jnp.concatenate([dst, jnp.full((pad,), N_NODES, jnp.int32)])
    src2 = jnp.stack([src, src + N_NODES]).reshape(NC, NS * n_sb, NB, CHUNK)
    dst2 = dst.reshape(NS * n_sb, NB, CHUNK)
    xh = jnp.concatenate([x[:, :DH], x[:, DH:]], axis=0)
    part, cnt = _sc_aggregate(xh, src2, dst2, n_sb)
    return _tc_finish(part, cnt, x, W_r)
